# Initial kernel scaffold; baseline (speedup 1.0000x reference)
import jax
import jax.numpy as jnp
from jax.experimental import pallas as pl


def kernel(feat, map_ctrs, actors, actor_ctrs, params, map_idcs, actor_idcs):
    raise NotImplementedError("write your pallas kernel here")


# speedup vs baseline: 26.6741x; 26.6741x over previous
import functools

import jax
import jax.numpy as jnp
from jax.experimental import pallas as pl

_EPS = 1e-5
_TH = 7.0
M_BLK = 200
E_BLK = 1024
N_ACT = 512


def _gn(x, g, b):
    mu = jnp.mean(x, axis=1, keepdims=True)
    xc = x - mu
    var = jnp.mean(xc * xc, axis=1, keepdims=True)
    return xc / jnp.sqrt(var + _EPS) * g + b


def _mm(x, w):
    return jax.lax.dot_general(x, w, (((1,), (1,)), ((), ())),
                               preferred_element_type=jnp.float32,
                               precision=jax.lax.Precision.HIGHEST)


def _layer_body(fuse_meta,
                agts_ref, mctr_ref, codes_ref, actr_ref, cw_ref,
                mw_ref, mg_ref, mb_ref,
                qw_ref, qg_ref, qb_ref, wq_ref,
                w1x_ref, w1y_ref, b1_ref, w2_ref, g2_ref, be2_ref,
                wd_ref, cg1_ref, cb1_ref, cw2_ref,
                agtw_ref, linw_ref, ling_ref, linb_ref,
                out_ref):
    x = agts_ref[...]
    if fuse_meta:
        x = jax.nn.relu(_gn(_mm(x, mw_ref[...]), mg_ref[...], mb_ref[...]))
    res = x
    q = jax.nn.relu(_gn(_mm(x, qw_ref[...]), qg_ref[...], qb_ref[...]))
    qW = _mm(q, wq_ref[...])
    agts0 = _mm(x, agtw_ref[...])

    codes = codes_ref[0, 0, :]
    li = codes // N_ACT
    aj = codes % N_ACT
    e_iota_m = jax.lax.broadcasted_iota(jnp.int32, (E_BLK, M_BLK), 1)
    oh_li = (e_iota_m == li[:, None]).astype(jnp.float32)
    e_iota_a = jax.lax.broadcasted_iota(jnp.int32, (E_BLK, N_ACT), 1)
    oh_aj = (e_iota_a == aj[:, None]).astype(jnp.float32)

    s_q = jnp.dot(oh_li, qW, preferred_element_type=jnp.float32,
                  precision=jax.lax.Precision.HIGHEST)
    mxy = jnp.dot(oh_li, mctr_ref[...], preferred_element_type=jnp.float32,
                  precision=jax.lax.Precision.HIGHEST)
    axy = jnp.dot(oh_aj, actr_ref[...], preferred_element_type=jnp.float32,
                  precision=jax.lax.Precision.HIGHEST)
    cw_e = jnp.dot(oh_aj, cw_ref[...], preferred_element_type=jnp.float32,
                  precision=jax.lax.Precision.HIGHEST)

    dist = mxy - axy
    dx = dist[:, 0:1]
    dy = dist[:, 1:2]
    h = jax.nn.relu(dx * w1x_ref[...] + dy * w1y_ref[...] + b1_ref[...])
    d = jax.nn.relu(_gn(_mm(h, w2_ref[...]), g2_ref[...], be2_ref[...]))
    pre = _mm(d, wd_ref[...]) + s_q + cw_e
    cc = jax.nn.relu(_gn(pre, cg1_ref[...], cb1_ref[...]))
    ce = _mm(cc, cw2_ref[...])

    m_iota_e = jax.lax.broadcasted_iota(jnp.int32, (M_BLK, E_BLK), 0)
    oh_t = (m_iota_e == li[None, :]).astype(jnp.float32)
    acc = agts0 + jnp.dot(oh_t, ce, preferred_element_type=jnp.float32,
                  precision=jax.lax.Precision.HIGHEST)

    x2 = jax.nn.relu(acc)
    y = _gn(_mm(x2, linw_ref[...]), ling_ref[...], linb_ref[...])
    out_ref[...] = jax.nn.relu(y + res)


def _cw_body(actors_ref, wc0_ref, wc1_ref, cw0_ref, cw1_ref):
    a = actors_ref[...]
    cw0_ref[...] = _mm(a, wc0_ref[...])
    cw1_ref[...] = _mm(a, wc1_ref[...])


def _row(shape):
    return pl.BlockSpec(shape, lambda i: (0,) * len(shape))


def _layer_call(fuse_meta, agts, mctr, codes, actr, cw, pars):
    n_map = agts.shape[0]
    nb = n_map // M_BLK
    in_specs = [
        pl.BlockSpec((M_BLK, 128), lambda i: (i, 0)),
        pl.BlockSpec((M_BLK, 2), lambda i: (i, 0)),
        pl.BlockSpec((1, 1, E_BLK), lambda i: (i, 0, 0)),
        _row((N_ACT, 2)),
        _row((N_ACT, 128)),
    ] + [_row(p.shape) for p in pars]
    return pl.pallas_call(
        functools.partial(_layer_body, fuse_meta),
        grid=(nb,),
        in_specs=in_specs,
        out_specs=pl.BlockSpec((M_BLK, 128), lambda i: (i, 0)),
        out_shape=jax.ShapeDtypeStruct((n_map, 128), jnp.float32),
    )(agts, mctr, codes, actr, cw, *pars)


def _att_pars(p):
    r = lambda v: v.reshape(1, -1)
    ctx_w1 = p['ctx_w1']
    return (
        p['query_w'], r(p['query_g']), r(p['query_b']), ctx_w1[:, 128:256],
        r(p['dist_w1'][:, 0]), r(p['dist_w1'][:, 1]), r(p['dist_b1']),
        p['dist_w2'], r(p['dist_g2']), r(p['dist_be2']),
        ctx_w1[:, 0:128], r(p['ctx_g1']), r(p['ctx_b1']), p['ctx_w2'],
        p['agt_w'], p['lin_w'], r(p['lin_g']), r(p['lin_b']),
    )


def kernel(feat, map_ctrs, actors, actor_ctrs, params, map_idcs, actor_idcs):
    n_map = feat.shape[0]
    nb = n_map // M_BLK

    diff = map_ctrs[:, None, :] - actor_ctrs[None, :, :]
    dist = jnp.sqrt((diff ** 2).sum(-1))
    mask = (dist <= _TH).reshape(nb, M_BLK * N_ACT)

    def comp(m):
        return jnp.nonzero(m, size=E_BLK, fill_value=M_BLK * N_ACT)[0]

    codes = jax.vmap(comp)(mask).astype(jnp.int32).reshape(nb, 1, E_BLK)

    cw0, cw1 = pl.pallas_call(
        _cw_body,
        out_shape=[jax.ShapeDtypeStruct((N_ACT, 128), jnp.float32)] * 2,
    )(actors, params['att0']['ctx_w1'][:, 256:],
      params['att1']['ctx_w1'][:, 256:])

    meta = (params['meta_w'], params['meta_g'].reshape(1, -1),
            params['meta_b'].reshape(1, -1))
    p0 = meta + _att_pars(params['att0'])
    p1 = meta + _att_pars(params['att1'])

    x = _layer_call(True, feat, map_ctrs, codes, actor_ctrs, cw0, p0)
    x = _layer_call(False, x, map_ctrs, codes, actor_ctrs, cw1, p1)
    return x


# speedup vs baseline: 49.3172x; 1.8489x over previous
import functools

import jax
import jax.numpy as jnp
from jax.experimental import pallas as pl

_EPS = 1e-5
_TH = 7.0
M_BLK = 200
E_BLK = 640
N_ACT = 512


def _gn(x, g, b):
    mu = jnp.mean(x, axis=1, keepdims=True)
    xc = x - mu
    var = jnp.mean(xc * xc, axis=1, keepdims=True)
    return xc / jnp.sqrt(var + _EPS) * g + b


def _dotT(x, w):
    return jax.lax.dot_general(x, w, (((1,), (1,)), ((), ())),
                               preferred_element_type=jnp.float32)


def _split(v):
    hi = v.astype(jnp.bfloat16).astype(jnp.float32)
    return hi, v - hi


def _mm(x, w):
    xh, xl = _split(x)
    wh, wl = _split(w)
    return _dotT(xh, wh) + (_dotT(xl, wh) + _dotT(xh, wl))


def _oh_mm(oh, v):
    vh, vl = _split(v)
    return (jnp.dot(oh, vh, preferred_element_type=jnp.float32)
            + jnp.dot(oh, vl, preferred_element_type=jnp.float32))


def _layer_body(fuse_meta,
                agts_ref, mctr_ref, codes_ref, actr_ref, cw_ref,
                mw_ref, mg_ref, mb_ref,
                qw_ref, qg_ref, qb_ref, wq_ref,
                w1x_ref, w1y_ref, b1_ref, w2_ref, g2_ref, be2_ref,
                wd_ref, cg1_ref, cb1_ref, cw2_ref,
                agtw_ref, linw_ref, ling_ref, linb_ref,
                out_ref):
    x = agts_ref[...]
    if fuse_meta:
        x = jax.nn.relu(_gn(_mm(x, mw_ref[...]), mg_ref[...], mb_ref[...]))
    res = x
    q = jax.nn.relu(_gn(_mm(x, qw_ref[...]), qg_ref[...], qb_ref[...]))
    qW = _mm(q, wq_ref[...])
    agts0 = _mm(x, agtw_ref[...])

    codes = codes_ref[0, 0, :]
    li = codes // N_ACT
    aj = codes % N_ACT
    e_iota_m = jax.lax.broadcasted_iota(jnp.int32, (E_BLK, M_BLK), 1)
    oh_li = (e_iota_m == li[:, None]).astype(jnp.float32)
    e_iota_a = jax.lax.broadcasted_iota(jnp.int32, (E_BLK, N_ACT), 1)
    oh_aj = (e_iota_a == aj[:, None]).astype(jnp.float32)

    s_q = _oh_mm(oh_li, qW)
    mxy = _oh_mm(oh_li, mctr_ref[...])
    axy = _oh_mm(oh_aj, actr_ref[...])
    cw_e = _oh_mm(oh_aj, cw_ref[...])

    dist = mxy - axy
    dx = dist[:, 0:1]
    dy = dist[:, 1:2]
    h = jax.nn.relu(dx * w1x_ref[...] + dy * w1y_ref[...] + b1_ref[...])
    d = jax.nn.relu(_gn(_mm(h, w2_ref[...]), g2_ref[...], be2_ref[...]))
    pre = _mm(d, wd_ref[...]) + s_q + cw_e
    cc = jax.nn.relu(_gn(pre, cg1_ref[...], cb1_ref[...]))
    ce = _mm(cc, cw2_ref[...])

    m_iota_e = jax.lax.broadcasted_iota(jnp.int32, (M_BLK, E_BLK), 0)
    oh_t = (m_iota_e == li[None, :]).astype(jnp.float32)
    acc = agts0 + _oh_mm(oh_t, ce)

    x2 = jax.nn.relu(acc)
    y = _gn(_mm(x2, linw_ref[...]), ling_ref[...], linb_ref[...])
    out_ref[...] = jax.nn.relu(y + res)


def _cw_body(actors_ref, wc0_ref, wc1_ref, cw0_ref, cw1_ref):
    a = actors_ref[...]
    cw0_ref[...] = _mm(a, wc0_ref[...])
    cw1_ref[...] = _mm(a, wc1_ref[...])


def _row(shape):
    return pl.BlockSpec(shape, lambda i: (0,) * len(shape))


def _layer_call(fuse_meta, agts, mctr, codes, actr, cw, pars):
    n_map = agts.shape[0]
    nb = n_map // M_BLK
    in_specs = [
        pl.BlockSpec((M_BLK, 128), lambda i: (i, 0)),
        pl.BlockSpec((M_BLK, 2), lambda i: (i, 0)),
        pl.BlockSpec((1, 1, E_BLK), lambda i: (i, 0, 0)),
        _row((N_ACT, 2)),
        _row((N_ACT, 128)),
    ] + [_row(p.shape) for p in pars]
    return pl.pallas_call(
        functools.partial(_layer_body, fuse_meta),
        grid=(nb,),
        in_specs=in_specs,
        out_specs=pl.BlockSpec((M_BLK, 128), lambda i: (i, 0)),
        out_shape=jax.ShapeDtypeStruct((n_map, 128), jnp.float32),
    )(agts, mctr, codes, actr, cw, *pars)


def _att_pars(p):
    r = lambda v: v.reshape(1, -1)
    ctx_w1 = p['ctx_w1']
    return (
        p['query_w'], r(p['query_g']), r(p['query_b']), ctx_w1[:, 128:256],
        r(p['dist_w1'][:, 0]), r(p['dist_w1'][:, 1]), r(p['dist_b1']),
        p['dist_w2'], r(p['dist_g2']), r(p['dist_be2']),
        ctx_w1[:, 0:128], r(p['ctx_g1']), r(p['ctx_b1']), p['ctx_w2'],
        p['agt_w'], p['lin_w'], r(p['lin_g']), r(p['lin_b']),
    )


def kernel(feat, map_ctrs, actors, actor_ctrs, params, map_idcs, actor_idcs):
    n_map = feat.shape[0]
    nb = n_map // M_BLK

    diff = map_ctrs[:, None, :] - actor_ctrs[None, :, :]
    dist = jnp.sqrt((diff ** 2).sum(-1))
    mask = (dist <= _TH).reshape(nb, M_BLK * N_ACT)

    def comp(m):
        return jnp.nonzero(m, size=E_BLK, fill_value=M_BLK * N_ACT)[0]

    codes = jax.vmap(comp)(mask).astype(jnp.int32).reshape(nb, 1, E_BLK)

    cw0, cw1 = pl.pallas_call(
        _cw_body,
        out_shape=[jax.ShapeDtypeStruct((N_ACT, 128), jnp.float32)] * 2,
    )(actors, params['att0']['ctx_w1'][:, 256:],
      params['att1']['ctx_w1'][:, 256:])

    meta = (params['meta_w'], params['meta_g'].reshape(1, -1),
            params['meta_b'].reshape(1, -1))
    p0 = meta + _att_pars(params['att0'])
    p1 = meta + _att_pars(params['att1'])

    x = _layer_call(True, feat, map_ctrs, codes, actor_ctrs, cw0, p0)
    x = _layer_call(False, x, map_ctrs, codes, actor_ctrs, cw1, p1)
    return x


# speedup vs baseline: 64.4239x; 1.3063x over previous
import functools

import jax
import jax.numpy as jnp
from jax.experimental import pallas as pl
from jax.experimental.pallas import tpu as pltpu
from jax.experimental.pallas import tpu_sc as plsc

_EPS = 1e-5
_TH = 7.0
M_BLK = 200
E_BLK = 640
N_ACT = 512


def _gn(x, g, b):
    mu = jnp.mean(x, axis=1, keepdims=True)
    xc = x - mu
    var = jnp.mean(xc * xc, axis=1, keepdims=True)
    return xc / jnp.sqrt(var + _EPS) * g + b


def _dotT(x, w):
    return jax.lax.dot_general(x, w, (((1,), (1,)), ((), ())),
                               preferred_element_type=jnp.float32)


def _split(v):
    hi = v.astype(jnp.bfloat16).astype(jnp.float32)
    return hi, v - hi


def _mm(x, w):
    xh, xl = _split(x)
    wh, wl = _split(w)
    return _dotT(xh, wh) + (_dotT(xl, wh) + _dotT(xh, wl))


def _oh_mm(oh, v):
    vh, vl = _split(v)
    return (jnp.dot(oh, vh, preferred_element_type=jnp.float32)
            + jnp.dot(oh, vl, preferred_element_type=jnp.float32))


_FILL = M_BLK * N_ACT
_NB = 100
_NW = 32
_REPS = (_NB + _NW - 1) // _NW
_CAP = E_BLK + 128


def _disc_body(mx_hbm, my_hbm, ax_hbm, ay_hbm, out_hbm,
               codes_v, mx_v, my_v, ax_v, ay_v):
    wid = jax.lax.axis_index("s") * 2 + jax.lax.axis_index("c")
    pltpu.sync_copy(ax_hbm, ax_v)
    pltpu.sync_copy(ay_hbm, ay_v)
    for rep in range(_REPS):
        b = wid + rep * _NW
        bc = jnp.minimum(b, _NB - 1)
        pltpu.sync_copy(mx_hbm.at[pl.ds(bc * M_BLK, M_BLK + 8)],
                        mx_v.at[pl.ds(0, M_BLK + 8)])
        pltpu.sync_copy(my_hbm.at[pl.ds(bc * M_BLK, M_BLK + 8)],
                        my_v.at[pl.ds(0, M_BLK + 8)])
        for i in range(_CAP // 16):
            codes_v[pl.ds(i * 16, 16)] = jnp.full((16,), _FILL, jnp.int32)

        def row(r, cnt):
            mx = mx_v[pl.ds(r, 16)][0]
            my = my_v[pl.ds(r, 16)][0]
            base = r * N_ACT

            def avec(jv, cnt):
                ax = ax_v[pl.ds(jv * 16, 16)]
                ay = ay_v[pl.ds(jv * 16, 16)]
                dx = ax - mx
                dy = ay - my
                d2 = dx * dx + dy * dy
                pred = d2 <= _TH * _TH
                inc = plsc.all_reduce_population_count(pred)[0]

                @pl.when(inc > 0)
                def _():
                    code = (base + jv * 16) + jax.lax.iota(jnp.int32, 16)
                    plsc.store_compressed(
                        codes_v.at[pl.ds(cnt, 16)], code, mask=pred)
                return cnt + inc

            for jv in range(N_ACT // 16):
                cnt = avec(jv, cnt)
            return cnt

        jax.lax.fori_loop(0, M_BLK, row, jnp.int32(0))
        pltpu.sync_copy(codes_v.at[pl.ds(0, E_BLK)], out_hbm.at[b])


def _discover(map_ctrs, actor_ctrs):
    mxp = jnp.pad(map_ctrs[:, 0], (0, 64))
    myp = jnp.pad(map_ctrs[:, 1], (0, 64))
    ax = actor_ctrs[:, 0]
    ay = actor_ctrs[:, 1]
    mesh = plsc.VectorSubcoreMesh(core_axis_name="c", subcore_axis_name="s")
    codes = pl.kernel(
        _disc_body,
        mesh=mesh,
        compiler_params=pltpu.CompilerParams(needs_layout_passes=False),
        out_type=jax.ShapeDtypeStruct((_NW * _REPS, E_BLK), jnp.int32),
        scratch_types=[
            pltpu.VMEM((_CAP,), jnp.int32),
            pltpu.VMEM((M_BLK + 16,), jnp.float32),
            pltpu.VMEM((M_BLK + 16,), jnp.float32),
            pltpu.VMEM((N_ACT,), jnp.float32),
            pltpu.VMEM((N_ACT,), jnp.float32),
        ],
    )(mxp, myp, ax, ay)
    return codes[:_NB].reshape(_NB, 1, E_BLK)


def _layer_body(fuse_meta,
                agts_ref, mctr_ref, codes_ref, actr_ref, cw_ref,
                mw_ref, mg_ref, mb_ref,
                qw_ref, qg_ref, qb_ref, wq_ref,
                w1x_ref, w1y_ref, b1_ref, w2_ref, g2_ref, be2_ref,
                wd_ref, cg1_ref, cb1_ref, cw2_ref,
                agtw_ref, linw_ref, ling_ref, linb_ref,
                out_ref):
    x = agts_ref[...]
    if fuse_meta:
        x = jax.nn.relu(_gn(_mm(x, mw_ref[...]), mg_ref[...], mb_ref[...]))
    res = x
    q = jax.nn.relu(_gn(_mm(x, qw_ref[...]), qg_ref[...], qb_ref[...]))
    qW = _mm(q, wq_ref[...])
    agts0 = _mm(x, agtw_ref[...])

    codes = codes_ref[0, 0, :]
    li = codes // N_ACT
    aj = codes % N_ACT
    e_iota_m = jax.lax.broadcasted_iota(jnp.int32, (E_BLK, M_BLK), 1)
    oh_li = (e_iota_m == li[:, None]).astype(jnp.float32)
    e_iota_a = jax.lax.broadcasted_iota(jnp.int32, (E_BLK, N_ACT), 1)
    oh_aj = (e_iota_a == aj[:, None]).astype(jnp.float32)

    s_q = _oh_mm(oh_li, qW)
    mxy = _oh_mm(oh_li, mctr_ref[...])
    axy = _oh_mm(oh_aj, actr_ref[...])
    cw_e = _oh_mm(oh_aj, cw_ref[...])

    dist = mxy - axy
    dx = dist[:, 0:1]
    dy = dist[:, 1:2]
    h = jax.nn.relu(dx * w1x_ref[...] + dy * w1y_ref[...] + b1_ref[...])
    d = jax.nn.relu(_gn(_mm(h, w2_ref[...]), g2_ref[...], be2_ref[...]))
    pre = _mm(d, wd_ref[...]) + s_q + cw_e
    cc = jax.nn.relu(_gn(pre, cg1_ref[...], cb1_ref[...]))
    ce = _mm(cc, cw2_ref[...])

    m_iota_e = jax.lax.broadcasted_iota(jnp.int32, (M_BLK, E_BLK), 0)
    oh_t = (m_iota_e == li[None, :]).astype(jnp.float32)
    acc = agts0 + _oh_mm(oh_t, ce)

    x2 = jax.nn.relu(acc)
    y = _gn(_mm(x2, linw_ref[...]), ling_ref[...], linb_ref[...])
    out_ref[...] = jax.nn.relu(y + res)


def _cw_body(actors_ref, wc0_ref, wc1_ref, cw0_ref, cw1_ref):
    a = actors_ref[...]
    cw0_ref[...] = _mm(a, wc0_ref[...])
    cw1_ref[...] = _mm(a, wc1_ref[...])


def _row(shape):
    return pl.BlockSpec(shape, lambda i: (0,) * len(shape))


def _layer_call(fuse_meta, agts, mctr, codes, actr, cw, pars):
    n_map = agts.shape[0]
    nb = n_map // M_BLK
    in_specs = [
        pl.BlockSpec((M_BLK, 128), lambda i: (i, 0)),
        pl.BlockSpec((M_BLK, 2), lambda i: (i, 0)),
        pl.BlockSpec((1, 1, E_BLK), lambda i: (i, 0, 0)),
        _row((N_ACT, 2)),
        _row((N_ACT, 128)),
    ] + [_row(p.shape) for p in pars]
    return pl.pallas_call(
        functools.partial(_layer_body, fuse_meta),
        grid=(nb,),
        in_specs=in_specs,
        out_specs=pl.BlockSpec((M_BLK, 128), lambda i: (i, 0)),
        out_shape=jax.ShapeDtypeStruct((n_map, 128), jnp.float32),
    )(agts, mctr, codes, actr, cw, *pars)


def _att_pars(p):
    r = lambda v: v.reshape(1, -1)
    ctx_w1 = p['ctx_w1']
    return (
        p['query_w'], r(p['query_g']), r(p['query_b']), ctx_w1[:, 128:256],
        r(p['dist_w1'][:, 0]), r(p['dist_w1'][:, 1]), r(p['dist_b1']),
        p['dist_w2'], r(p['dist_g2']), r(p['dist_be2']),
        ctx_w1[:, 0:128], r(p['ctx_g1']), r(p['ctx_b1']), p['ctx_w2'],
        p['agt_w'], p['lin_w'], r(p['lin_g']), r(p['lin_b']),
    )


def kernel(feat, map_ctrs, actors, actor_ctrs, params, map_idcs, actor_idcs):
    n_map = feat.shape[0]
    nb = n_map // M_BLK

    codes = _discover(map_ctrs, actor_ctrs)

    cw0, cw1 = pl.pallas_call(
        _cw_body,
        out_shape=[jax.ShapeDtypeStruct((N_ACT, 128), jnp.float32)] * 2,
    )(actors, params['att0']['ctx_w1'][:, 256:],
      params['att1']['ctx_w1'][:, 256:])

    meta = (params['meta_w'], params['meta_g'].reshape(1, -1),
            params['meta_b'].reshape(1, -1))
    p0 = meta + _att_pars(params['att0'])
    p1 = meta + _att_pars(params['att1'])

    x = _layer_call(True, feat, map_ctrs, codes, actor_ctrs, cw0, p0)
    x = _layer_call(False, x, map_ctrs, codes, actor_ctrs, cw1, p1)
    return x


# speedup vs baseline: 91.1372x; 1.4147x over previous
import functools

import jax
import jax.numpy as jnp
from jax.experimental import pallas as pl
from jax.experimental.pallas import tpu as pltpu
from jax.experimental.pallas import tpu_sc as plsc

_EPS = 1e-5
_TH = 7.0
M_BLK = 200
E_BLK = 640
N_ACT = 512


def _gn(x, g, b):
    mu = jnp.mean(x, axis=1, keepdims=True)
    xc = x - mu
    var = jnp.mean(xc * xc, axis=1, keepdims=True)
    return xc / jnp.sqrt(var + _EPS) * g + b


def _dotT(x, w):
    return jax.lax.dot_general(x, w, (((1,), (1,)), ((), ())),
                               preferred_element_type=jnp.float32)


def _split(v):
    hi = v.astype(jnp.bfloat16).astype(jnp.float32)
    return hi, v - hi


def _mm(x, w):
    xh, xl = _split(x)
    wh, wl = _split(w)
    return _dotT(xh, wh) + (_dotT(xl, wh) + _dotT(xh, wl))


def _oh_mm(oh, v):
    vh, vl = _split(v)
    return (jnp.dot(oh, vh, preferred_element_type=jnp.float32)
            + jnp.dot(oh, vl, preferred_element_type=jnp.float32))


_FILL = M_BLK * N_ACT
_NB = 100
_NW = 32
_REPS = (_NB + _NW - 1) // _NW
_CAP = E_BLK + 128


def _disc_body(mx_hbm, my_hbm, ax_hbm, ay_hbm, jid_hbm, out_hbm,
               codes_v, mx_v, my_v, ax_v, ay_v, jid_v):
    wid = jax.lax.axis_index("s") * 2 + jax.lax.axis_index("c")
    pltpu.sync_copy(ax_hbm, ax_v)
    pltpu.sync_copy(ay_hbm, ay_v)
    pltpu.sync_copy(jid_hbm, jid_v)
    def rep_body(rep, _):
        b = wid + rep * _NW
        bc = jnp.minimum(b, _NB - 1)
        pltpu.sync_copy(mx_hbm.at[pl.ds(bc * M_BLK, M_BLK + 8)],
                        mx_v.at[pl.ds(0, M_BLK + 8)])
        pltpu.sync_copy(my_hbm.at[pl.ds(bc * M_BLK, M_BLK + 8)],
                        my_v.at[pl.ds(0, M_BLK + 8)])

        def init(i, _):
            codes_v[pl.ds(i * 16, 16)] = jnp.full((16,), _FILL, jnp.int32)
            return 0

        jax.lax.fori_loop(0, _CAP // 16, init, 0)

        def vstep(v, cnt):
            mxv = mx_v[pl.ds(v * 16, 16)]
            myv = my_v[pl.ds(v * 16, 16)]
            lov = jnp.zeros((16,), jnp.int32)
            upv = jnp.zeros((16,), jnp.int32)
            hi1 = jnp.full((16,), N_ACT, jnp.int32)
            hi2 = jnp.full((16,), N_ACT, jnp.int32)
            xlo = mxv - _TH
            xhi = mxv + _TH
            for _u in range(9):
                mid1 = (lov + hi1) // 2
                val1 = plsc.load_gather(ax_v, [mid1])
                c1 = val1 < xlo
                lov = jnp.where(c1, mid1 + 1, lov)
                hi1 = jnp.where(c1, hi1, mid1)
                mid2 = (upv + hi2) // 2
                val2 = plsc.load_gather(ax_v, [mid2])
                c2 = val2 <= xhi
                upv = jnp.where(c2, mid2 + 1, upv)
                hi2 = jnp.where(c2, hi2, mid2)
            tmax = plsc.cummax(upv - lov)[15]
            rowbase = (v * jnp.int32(16 * N_ACT)
                       + jax.lax.iota(jnp.int32, 16) * N_ACT)

            def tstep(t, cnt):
                idx = lov + t
                valid = idx < upv
                idxc = jnp.minimum(idx, N_ACT - 1)
                axg = plsc.load_gather(ax_v, [idxc])
                ayg = plsc.load_gather(ay_v, [idxc])
                dx = axg - mxv
                dy = ayg - myv
                d2 = dx * dx + dy * dy
                pred = jnp.logical_and(d2 <= _TH * _TH, valid)
                inc = plsc.all_reduce_population_count(pred)[0]

                @pl.when(inc > 0)
                def _():
                    jg = plsc.load_gather(jid_v, [idxc])
                    code = rowbase + jg
                    plsc.store_compressed(
                        codes_v.at[pl.ds(cnt, 16)], code, mask=pred)
                return cnt + inc

            return jax.lax.fori_loop(0, tmax, tstep, cnt)

        jax.lax.fori_loop(0, (M_BLK + 15) // 16, vstep, jnp.int32(0))
        pltpu.sync_copy(codes_v.at[pl.ds(0, E_BLK)], out_hbm.at[b])
        return 0

    jax.lax.fori_loop(0, _REPS, rep_body, 0)


def _discover(map_ctrs, actor_ctrs):
    mxp = jnp.pad(map_ctrs[:, 0], (0, 64), constant_values=1e9)
    myp = jnp.pad(map_ctrs[:, 1], (0, 64), constant_values=1e9)
    order = jnp.argsort(actor_ctrs[:, 0])
    ax = actor_ctrs[order, 0]
    ay = actor_ctrs[order, 1]
    jid = order.astype(jnp.int32)
    mesh = plsc.VectorSubcoreMesh(core_axis_name="c", subcore_axis_name="s")
    codes = pl.kernel(
        _disc_body,
        mesh=mesh,
        compiler_params=pltpu.CompilerParams(needs_layout_passes=False),
        out_type=jax.ShapeDtypeStruct((_NW * _REPS, E_BLK), jnp.int32),
        scratch_types=[
            pltpu.VMEM((_CAP,), jnp.int32),
            pltpu.VMEM((M_BLK + 16,), jnp.float32),
            pltpu.VMEM((M_BLK + 16,), jnp.float32),
            pltpu.VMEM((N_ACT,), jnp.float32),
            pltpu.VMEM((N_ACT,), jnp.float32),
            pltpu.VMEM((N_ACT,), jnp.int32),
        ],
    )(mxp, myp, ax, ay, jid)
    return codes[:_NB].reshape(_NB, 1, E_BLK)


def _layer_body(fuse_meta,
                agts_ref, mctr_ref, codes_ref, actr_ref, cw_ref,
                mw_ref, mg_ref, mb_ref,
                qw_ref, qg_ref, qb_ref, wq_ref,
                w1x_ref, w1y_ref, b1_ref, w2_ref, g2_ref, be2_ref,
                wd_ref, cg1_ref, cb1_ref, cw2_ref,
                agtw_ref, linw_ref, ling_ref, linb_ref,
                out_ref):
    x = agts_ref[...]
    if fuse_meta:
        x = jax.nn.relu(_gn(_mm(x, mw_ref[...]), mg_ref[...], mb_ref[...]))
    res = x
    q = jax.nn.relu(_gn(_mm(x, qw_ref[...]), qg_ref[...], qb_ref[...]))
    qW = _mm(q, wq_ref[...])
    agts0 = _mm(x, agtw_ref[...])

    codes = codes_ref[0, 0, :]
    li = codes // N_ACT
    aj = codes % N_ACT
    e_iota_m = jax.lax.broadcasted_iota(jnp.int32, (E_BLK, M_BLK), 1)
    oh_li = (e_iota_m == li[:, None]).astype(jnp.float32)
    e_iota_a = jax.lax.broadcasted_iota(jnp.int32, (E_BLK, N_ACT), 1)
    oh_aj = (e_iota_a == aj[:, None]).astype(jnp.float32)

    s_q = _oh_mm(oh_li, qW)
    mxy = _oh_mm(oh_li, mctr_ref[...])
    axy = _oh_mm(oh_aj, actr_ref[...])
    cw_e = _oh_mm(oh_aj, cw_ref[...])

    dist = mxy - axy
    dx = dist[:, 0:1]
    dy = dist[:, 1:2]
    h = jax.nn.relu(dx * w1x_ref[...] + dy * w1y_ref[...] + b1_ref[...])
    d = jax.nn.relu(_gn(_mm(h, w2_ref[...]), g2_ref[...], be2_ref[...]))
    pre = _mm(d, wd_ref[...]) + s_q + cw_e
    cc = jax.nn.relu(_gn(pre, cg1_ref[...], cb1_ref[...]))
    ce = _mm(cc, cw2_ref[...])

    m_iota_e = jax.lax.broadcasted_iota(jnp.int32, (M_BLK, E_BLK), 0)
    oh_t = (m_iota_e == li[None, :]).astype(jnp.float32)
    acc = agts0 + _oh_mm(oh_t, ce)

    x2 = jax.nn.relu(acc)
    y = _gn(_mm(x2, linw_ref[...]), ling_ref[...], linb_ref[...])
    out_ref[...] = jax.nn.relu(y + res)


def _cw_body(actors_ref, wc0_ref, wc1_ref, cw0_ref, cw1_ref):
    a = actors_ref[...]
    cw0_ref[...] = _mm(a, wc0_ref[...])
    cw1_ref[...] = _mm(a, wc1_ref[...])


def _row(shape):
    return pl.BlockSpec(shape, lambda i: (0,) * len(shape))


def _layer_call(fuse_meta, agts, mctr, codes, actr, cw, pars):
    n_map = agts.shape[0]
    nb = n_map // M_BLK
    in_specs = [
        pl.BlockSpec((M_BLK, 128), lambda i: (i, 0)),
        pl.BlockSpec((M_BLK, 2), lambda i: (i, 0)),
        pl.BlockSpec((1, 1, E_BLK), lambda i: (i, 0, 0)),
        _row((N_ACT, 2)),
        _row((N_ACT, 128)),
    ] + [_row(p.shape) for p in pars]
    return pl.pallas_call(
        functools.partial(_layer_body, fuse_meta),
        grid=(nb,),
        in_specs=in_specs,
        out_specs=pl.BlockSpec((M_BLK, 128), lambda i: (i, 0)),
        out_shape=jax.ShapeDtypeStruct((n_map, 128), jnp.float32),
    )(agts, mctr, codes, actr, cw, *pars)


def _att_pars(p):
    r = lambda v: v.reshape(1, -1)
    ctx_w1 = p['ctx_w1']
    return (
        p['query_w'], r(p['query_g']), r(p['query_b']), ctx_w1[:, 128:256],
        r(p['dist_w1'][:, 0]), r(p['dist_w1'][:, 1]), r(p['dist_b1']),
        p['dist_w2'], r(p['dist_g2']), r(p['dist_be2']),
        ctx_w1[:, 0:128], r(p['ctx_g1']), r(p['ctx_b1']), p['ctx_w2'],
        p['agt_w'], p['lin_w'], r(p['lin_g']), r(p['lin_b']),
    )


def kernel(feat, map_ctrs, actors, actor_ctrs, params, map_idcs, actor_idcs):
    n_map = feat.shape[0]
    nb = n_map // M_BLK

    codes = _discover(map_ctrs, actor_ctrs)

    cw0, cw1 = pl.pallas_call(
        _cw_body,
        out_shape=[jax.ShapeDtypeStruct((N_ACT, 128), jnp.float32)] * 2,
    )(actors, params['att0']['ctx_w1'][:, 256:],
      params['att1']['ctx_w1'][:, 256:])

    meta = (params['meta_w'], params['meta_g'].reshape(1, -1),
            params['meta_b'].reshape(1, -1))
    p0 = meta + _att_pars(params['att0'])
    p1 = meta + _att_pars(params['att1'])

    x = _layer_call(True, feat, map_ctrs, codes, actor_ctrs, cw0, p0)
    x = _layer_call(False, x, map_ctrs, codes, actor_ctrs, cw1, p1)
    return x


# speedup vs baseline: 91.4718x; 1.0037x over previous
import functools

import jax
import jax.numpy as jnp
from jax.experimental import pallas as pl
from jax.experimental.pallas import tpu as pltpu
from jax.experimental.pallas import tpu_sc as plsc

_EPS = 1e-5
_TH = 7.0
M_BLK = 200
E_BLK = 640
N_ACT = 512


def _gn(x, g, b):
    mu = jnp.mean(x, axis=1, keepdims=True)
    xc = x - mu
    var = jnp.mean(xc * xc, axis=1, keepdims=True)
    return xc / jnp.sqrt(var + _EPS) * g + b


def _dotT(x, w):
    return jax.lax.dot_general(x, w, (((1,), (1,)), ((), ())),
                               preferred_element_type=jnp.float32)


def _split(v):
    hi = v.astype(jnp.bfloat16).astype(jnp.float32)
    return hi, v - hi


def _mm(x, w):
    xh, xl = _split(x)
    wh, wl = _split(w)
    b = jnp.bfloat16
    return (_dotT(xh.astype(b), wh.astype(b))
            + (_dotT(xl.astype(b), wh.astype(b))
               + _dotT(xh.astype(b), wl.astype(b))))


def _oh_mm(oh, v):
    vh, vl = _split(v)
    b = jnp.bfloat16
    return (jnp.dot(oh, vh.astype(b), preferred_element_type=jnp.float32)
            + jnp.dot(oh, vl.astype(b), preferred_element_type=jnp.float32))


def _oh_tmm(oh, v):
    b = jnp.bfloat16
    dn = (((0,), (0,)), ((), ()))
    vh, vl = _split(v)
    return (jax.lax.dot_general(oh, vh.astype(b), dn,
                                preferred_element_type=jnp.float32)
            + jax.lax.dot_general(oh, vl.astype(b), dn,
                                  preferred_element_type=jnp.float32))


_FILL = M_BLK * N_ACT
_NB = 100
_NW = 32
_REPS = (_NB + _NW - 1) // _NW
_CAP = E_BLK + 128


def _disc_body(mx_hbm, my_hbm, ax_hbm, ay_hbm, jid_hbm, out_hbm,
               codes_v, mx_v, my_v, ax_v, ay_v, jid_v):
    wid = jax.lax.axis_index("s") * 2 + jax.lax.axis_index("c")
    pltpu.sync_copy(ax_hbm, ax_v)
    pltpu.sync_copy(ay_hbm, ay_v)
    pltpu.sync_copy(jid_hbm, jid_v)
    def rep_body(rep, _):
        b = wid + rep * _NW
        bc = jnp.minimum(b, _NB - 1)
        pltpu.sync_copy(mx_hbm.at[pl.ds(bc * M_BLK, M_BLK + 8)],
                        mx_v.at[pl.ds(0, M_BLK + 8)])
        pltpu.sync_copy(my_hbm.at[pl.ds(bc * M_BLK, M_BLK + 8)],
                        my_v.at[pl.ds(0, M_BLK + 8)])

        def init(i, _):
            codes_v[pl.ds(i * 16, 16)] = jnp.full((16,), _FILL, jnp.int32)
            return 0

        jax.lax.fori_loop(0, _CAP // 16, init, 0)

        def vstep(v, cnt):
            mxv = mx_v[pl.ds(v * 16, 16)]
            myv = my_v[pl.ds(v * 16, 16)]
            lov = jnp.zeros((16,), jnp.int32)
            upv = jnp.zeros((16,), jnp.int32)
            hi1 = jnp.full((16,), N_ACT, jnp.int32)
            hi2 = jnp.full((16,), N_ACT, jnp.int32)
            xlo = mxv - _TH
            xhi = mxv + _TH
            for _u in range(9):
                mid1 = (lov + hi1) // 2
                val1 = plsc.load_gather(ax_v, [mid1])
                c1 = val1 < xlo
                lov = jnp.where(c1, mid1 + 1, lov)
                hi1 = jnp.where(c1, hi1, mid1)
                mid2 = (upv + hi2) // 2
                val2 = plsc.load_gather(ax_v, [mid2])
                c2 = val2 <= xhi
                upv = jnp.where(c2, mid2 + 1, upv)
                hi2 = jnp.where(c2, hi2, mid2)
            tmax = plsc.cummax(upv - lov)[15]
            rowbase = (v * jnp.int32(16 * N_ACT)
                       + jax.lax.iota(jnp.int32, 16) * N_ACT)

            def tstep(t, cnt):
                idx = lov + t
                valid = idx < upv
                idxc = jnp.minimum(idx, N_ACT - 1)
                axg = plsc.load_gather(ax_v, [idxc])
                ayg = plsc.load_gather(ay_v, [idxc])
                dx = axg - mxv
                dy = ayg - myv
                d2 = dx * dx + dy * dy
                pred = jnp.logical_and(d2 <= _TH * _TH, valid)
                inc = plsc.all_reduce_population_count(pred)[0]

                @pl.when(inc > 0)
                def _():
                    jg = plsc.load_gather(jid_v, [idxc])
                    code = rowbase + jg
                    plsc.store_compressed(
                        codes_v.at[pl.ds(cnt, 16)], code, mask=pred)
                return cnt + inc

            return jax.lax.fori_loop(0, tmax, tstep, cnt)

        jax.lax.fori_loop(0, (M_BLK + 15) // 16, vstep, jnp.int32(0))
        pltpu.sync_copy(codes_v.at[pl.ds(0, E_BLK)], out_hbm.at[b])
        return 0

    jax.lax.fori_loop(0, _REPS, rep_body, 0)


def _discover(map_ctrs, actor_ctrs):
    mxp = jnp.pad(map_ctrs[:, 0], (0, 64), constant_values=1e9)
    myp = jnp.pad(map_ctrs[:, 1], (0, 64), constant_values=1e9)
    order = jnp.argsort(actor_ctrs[:, 0])
    ax = actor_ctrs[order, 0]
    ay = actor_ctrs[order, 1]
    jid = order.astype(jnp.int32)
    mesh = plsc.VectorSubcoreMesh(core_axis_name="c", subcore_axis_name="s")
    codes = pl.kernel(
        _disc_body,
        mesh=mesh,
        compiler_params=pltpu.CompilerParams(needs_layout_passes=False),
        out_type=jax.ShapeDtypeStruct((_NW * _REPS, E_BLK), jnp.int32),
        scratch_types=[
            pltpu.VMEM((_CAP,), jnp.int32),
            pltpu.VMEM((M_BLK + 16,), jnp.float32),
            pltpu.VMEM((M_BLK + 16,), jnp.float32),
            pltpu.VMEM((N_ACT,), jnp.float32),
            pltpu.VMEM((N_ACT,), jnp.float32),
            pltpu.VMEM((N_ACT,), jnp.int32),
        ],
    )(mxp, myp, ax, ay, jid)
    return codes[:_NB].reshape(_NB, 1, E_BLK)


def _layer_body(fuse_meta,
                agts_ref, mctr_ref, codes_ref, actr_ref, cw_ref,
                mw_ref, mg_ref, mb_ref,
                qw_ref, qg_ref, qb_ref, wq_ref,
                w1x_ref, w1y_ref, b1_ref, w2_ref, g2_ref, be2_ref,
                wd_ref, cg1_ref, cb1_ref, cw2_ref,
                agtw_ref, linw_ref, ling_ref, linb_ref,
                out_ref):
    x = agts_ref[...]
    if fuse_meta:
        x = jax.nn.relu(_gn(_mm(x, mw_ref[...]), mg_ref[...], mb_ref[...]))
    res = x
    q = jax.nn.relu(_gn(_mm(x, qw_ref[...]), qg_ref[...], qb_ref[...]))
    qW = _mm(q, wq_ref[...])
    agts0 = _mm(x, agtw_ref[...])

    codes = codes_ref[0, 0, :]
    li = codes // N_ACT
    aj = codes % N_ACT
    e_iota_m = jax.lax.broadcasted_iota(jnp.int32, (E_BLK, M_BLK), 1)
    oh_li = (e_iota_m == li[:, None]).astype(jnp.bfloat16)
    e_iota_a = jax.lax.broadcasted_iota(jnp.int32, (E_BLK, N_ACT), 1)
    oh_aj = (e_iota_a == aj[:, None]).astype(jnp.bfloat16)

    s_q = _oh_mm(oh_li, qW)
    mxy = _oh_mm(oh_li, mctr_ref[...])
    axy = _oh_mm(oh_aj, actr_ref[...])
    cw_e = _oh_mm(oh_aj, cw_ref[...])

    dist = mxy - axy
    dx = dist[:, 0:1]
    dy = dist[:, 1:2]
    h = jax.nn.relu(dx * w1x_ref[...] + dy * w1y_ref[...] + b1_ref[...])
    d = jax.nn.relu(_gn(_mm(h, w2_ref[...]), g2_ref[...], be2_ref[...]))
    pre = _mm(d, wd_ref[...]) + s_q + cw_e
    cc = jax.nn.relu(_gn(pre, cg1_ref[...], cb1_ref[...]))
    ce = _mm(cc, cw2_ref[...])

    acc = agts0 + _oh_tmm(oh_li, ce)

    x2 = jax.nn.relu(acc)
    y = _gn(_mm(x2, linw_ref[...]), ling_ref[...], linb_ref[...])
    out_ref[...] = jax.nn.relu(y + res)


def _cw_body(actors_ref, wc0_ref, wc1_ref, cw0_ref, cw1_ref):
    a = actors_ref[...]
    cw0_ref[...] = _mm(a, wc0_ref[...])
    cw1_ref[...] = _mm(a, wc1_ref[...])


def _row(shape):
    return pl.BlockSpec(shape, lambda i: (0,) * len(shape))


def _layer_call(fuse_meta, agts, mctr, codes, actr, cw, pars):
    n_map = agts.shape[0]
    nb = n_map // M_BLK
    in_specs = [
        pl.BlockSpec((M_BLK, 128), lambda i: (i, 0)),
        pl.BlockSpec((M_BLK, 2), lambda i: (i, 0)),
        pl.BlockSpec((1, 1, E_BLK), lambda i: (i, 0, 0)),
        _row((N_ACT, 2)),
        _row((N_ACT, 128)),
    ] + [_row(p.shape) for p in pars]
    return pl.pallas_call(
        functools.partial(_layer_body, fuse_meta),
        grid=(nb,),
        in_specs=in_specs,
        out_specs=pl.BlockSpec((M_BLK, 128), lambda i: (i, 0)),
        out_shape=jax.ShapeDtypeStruct((n_map, 128), jnp.float32),
    )(agts, mctr, codes, actr, cw, *pars)


def _att_pars(p):
    r = lambda v: v.reshape(1, -1)
    ctx_w1 = p['ctx_w1']
    return (
        p['query_w'], r(p['query_g']), r(p['query_b']), ctx_w1[:, 128:256],
        r(p['dist_w1'][:, 0]), r(p['dist_w1'][:, 1]), r(p['dist_b1']),
        p['dist_w2'], r(p['dist_g2']), r(p['dist_be2']),
        ctx_w1[:, 0:128], r(p['ctx_g1']), r(p['ctx_b1']), p['ctx_w2'],
        p['agt_w'], p['lin_w'], r(p['lin_g']), r(p['lin_b']),
    )


def kernel(feat, map_ctrs, actors, actor_ctrs, params, map_idcs, actor_idcs):
    n_map = feat.shape[0]
    nb = n_map // M_BLK

    codes = _discover(map_ctrs, actor_ctrs)

    cw0, cw1 = pl.pallas_call(
        _cw_body,
        out_shape=[jax.ShapeDtypeStruct((N_ACT, 128), jnp.float32)] * 2,
    )(actors, params['att0']['ctx_w1'][:, 256:],
      params['att1']['ctx_w1'][:, 256:])

    meta = (params['meta_w'], params['meta_g'].reshape(1, -1),
            params['meta_b'].reshape(1, -1))
    p0 = meta + _att_pars(params['att0'])
    p1 = meta + _att_pars(params['att1'])

    x = _layer_call(True, feat, map_ctrs, codes, actor_ctrs, cw0, p0)
    x = _layer_call(False, x, map_ctrs, codes, actor_ctrs, cw1, p1)
    return x


# speedup vs baseline: 109.5303x; 1.1974x over previous
import functools

import jax
import jax.numpy as jnp
from jax.experimental import pallas as pl
from jax.experimental.pallas import tpu as pltpu
from jax.experimental.pallas import tpu_sc as plsc

_EPS = 1e-5
_TH = 7.0
M_BLK = 200
E_BLK = 640
N_ACT = 512


def _gn(x, g, b):
    mu = jnp.mean(x, axis=1, keepdims=True)
    xc = x - mu
    var = jnp.mean(xc * xc, axis=1, keepdims=True)
    return xc / jnp.sqrt(var + _EPS) * g + b


def _dotT(x, w):
    return jax.lax.dot_general(x, w, (((1,), (1,)), ((), ())),
                               preferred_element_type=jnp.float32)


def _split(v):
    hi = v.astype(jnp.bfloat16).astype(jnp.float32)
    return hi, v - hi


def _mm(x, w):
    xh, xl = _split(x)
    wh, wl = _split(w)
    b = jnp.bfloat16
    return (_dotT(xh.astype(b), wh.astype(b))
            + (_dotT(xl.astype(b), wh.astype(b))
               + _dotT(xh.astype(b), wl.astype(b))))


def _oh_mm(oh, v):
    vh, vl = _split(v)
    b = jnp.bfloat16
    return (jnp.dot(oh, vh.astype(b), preferred_element_type=jnp.float32)
            + jnp.dot(oh, vl.astype(b), preferred_element_type=jnp.float32))


def _oh_tmm(oh, v):
    b = jnp.bfloat16
    dn = (((0,), (0,)), ((), ()))
    vh, vl = _split(v)
    return (jax.lax.dot_general(oh, vh.astype(b), dn,
                                preferred_element_type=jnp.float32)
            + jax.lax.dot_general(oh, vl.astype(b), dn,
                                  preferred_element_type=jnp.float32))


_FILL = M_BLK * N_ACT
_NB = 100
_NW = 32
_REPS = (_NB + _NW - 1) // _NW
_CAP = E_BLK + 128


def _disc_body(mx_hbm, my_hbm, ax_hbm, ay_hbm, jid_hbm,
               out_hbm, dxo_hbm, dyo_hbm,
               codes_v, dx_v, dy_v, mx_v, my_v, ax_v, ay_v, jid_v):
    wid = jax.lax.axis_index("s") * 2 + jax.lax.axis_index("c")
    pltpu.sync_copy(ax_hbm, ax_v)
    pltpu.sync_copy(ay_hbm, ay_v)
    pltpu.sync_copy(jid_hbm, jid_v)
    def rep_body(rep, _):
        b = wid + rep * _NW
        bc = jnp.minimum(b, _NB - 1)
        pltpu.sync_copy(mx_hbm.at[pl.ds(bc * M_BLK, M_BLK + 8)],
                        mx_v.at[pl.ds(0, M_BLK + 8)])
        pltpu.sync_copy(my_hbm.at[pl.ds(bc * M_BLK, M_BLK + 8)],
                        my_v.at[pl.ds(0, M_BLK + 8)])

        def init(i, _):
            codes_v[pl.ds(i * 16, 16)] = jnp.full((16,), _FILL, jnp.int32)
            dx_v[pl.ds(i * 16, 16)] = jnp.zeros((16,), jnp.float32)
            dy_v[pl.ds(i * 16, 16)] = jnp.zeros((16,), jnp.float32)
            return 0

        jax.lax.fori_loop(0, _CAP // 16, init, 0)

        def vstep(v, cnt):
            mxv = mx_v[pl.ds(v * 16, 16)]
            myv = my_v[pl.ds(v * 16, 16)]
            lov = jnp.zeros((16,), jnp.int32)
            upv = jnp.zeros((16,), jnp.int32)
            hi1 = jnp.full((16,), N_ACT, jnp.int32)
            hi2 = jnp.full((16,), N_ACT, jnp.int32)
            xlo = mxv - _TH
            xhi = mxv + _TH
            for _u in range(9):
                mid1 = (lov + hi1) // 2
                val1 = plsc.load_gather(ax_v, [mid1])
                c1 = val1 < xlo
                lov = jnp.where(c1, mid1 + 1, lov)
                hi1 = jnp.where(c1, hi1, mid1)
                mid2 = (upv + hi2) // 2
                val2 = plsc.load_gather(ax_v, [mid2])
                c2 = val2 <= xhi
                upv = jnp.where(c2, mid2 + 1, upv)
                hi2 = jnp.where(c2, hi2, mid2)
            tmax = plsc.cummax(upv - lov)[15]
            rowbase = (v * jnp.int32(16 * N_ACT)
                       + jax.lax.iota(jnp.int32, 16) * N_ACT)

            def tstep(t, cnt):
                idx = lov + t
                valid = idx < upv
                idxc = jnp.minimum(idx, N_ACT - 1)
                axg = plsc.load_gather(ax_v, [idxc])
                ayg = plsc.load_gather(ay_v, [idxc])
                dx = axg - mxv
                dy = ayg - myv
                d2 = dx * dx + dy * dy
                pred = jnp.logical_and(d2 <= _TH * _TH, valid)
                inc = plsc.all_reduce_population_count(pred)[0]

                @pl.when(inc > 0)
                def _():
                    jg = plsc.load_gather(jid_v, [idxc])
                    code = rowbase + jg
                    plsc.store_compressed(
                        codes_v.at[pl.ds(cnt, 16)], code, mask=pred)
                    plsc.store_compressed(
                        dx_v.at[pl.ds(cnt, 16)], -dx, mask=pred)
                    plsc.store_compressed(
                        dy_v.at[pl.ds(cnt, 16)], -dy, mask=pred)
                return cnt + inc

            return jax.lax.fori_loop(0, tmax, tstep, cnt)

        jax.lax.fori_loop(0, (M_BLK + 15) // 16, vstep, jnp.int32(0))
        pltpu.sync_copy(codes_v.at[pl.ds(0, E_BLK)], out_hbm.at[b])
        pltpu.sync_copy(dx_v.at[pl.ds(0, E_BLK)], dxo_hbm.at[b])
        pltpu.sync_copy(dy_v.at[pl.ds(0, E_BLK)], dyo_hbm.at[b])
        return 0

    jax.lax.fori_loop(0, _REPS, rep_body, 0)


def _discover(map_ctrs, actor_ctrs):
    mxp = jnp.pad(map_ctrs[:, 0], (0, 64), constant_values=1e9)
    myp = jnp.pad(map_ctrs[:, 1], (0, 64), constant_values=1e9)
    order = jnp.argsort(actor_ctrs[:, 0])
    ax = actor_ctrs[order, 0]
    ay = actor_ctrs[order, 1]
    jid = order.astype(jnp.int32)
    mesh = plsc.VectorSubcoreMesh(core_axis_name="c", subcore_axis_name="s")
    codes = pl.kernel(
        _disc_body,
        mesh=mesh,
        compiler_params=pltpu.CompilerParams(needs_layout_passes=False),
        out_type=[jax.ShapeDtypeStruct((_NW * _REPS, E_BLK), jnp.int32),
                  jax.ShapeDtypeStruct((_NW * _REPS, E_BLK), jnp.float32),
                  jax.ShapeDtypeStruct((_NW * _REPS, E_BLK), jnp.float32)],
        scratch_types=[
            pltpu.VMEM((_CAP,), jnp.int32),
            pltpu.VMEM((_CAP,), jnp.float32),
            pltpu.VMEM((_CAP,), jnp.float32),
            pltpu.VMEM((M_BLK + 16,), jnp.float32),
            pltpu.VMEM((M_BLK + 16,), jnp.float32),
            pltpu.VMEM((N_ACT,), jnp.float32),
            pltpu.VMEM((N_ACT,), jnp.float32),
            pltpu.VMEM((N_ACT,), jnp.int32),
        ],
    )(mxp, myp, ax, ay, jid)
    codes, dxs, dys = codes
    return (codes[:_NB].reshape(_NB, 1, E_BLK),
            dxs[:_NB].reshape(_NB, 1, E_BLK),
            dys[:_NB].reshape(_NB, 1, E_BLK))


def _layer_body(fuse_meta,
                agts_ref, codes_ref, dx_ref, dy_ref, cw_ref,
                mw_ref, mg_ref, mb_ref,
                qw_ref, qg_ref, qb_ref, wq_ref,
                w1x_ref, w1y_ref, b1_ref, w2_ref, g2_ref, be2_ref,
                wd_ref, cg1_ref, cb1_ref, cw2_ref,
                agtw_ref, linw_ref, ling_ref, linb_ref,
                out_ref):
    x = agts_ref[...]
    if fuse_meta:
        x = jax.nn.relu(_gn(_mm(x, mw_ref[...]), mg_ref[...], mb_ref[...]))
    res = x
    q = jax.nn.relu(_gn(_mm(x, qw_ref[...]), qg_ref[...], qb_ref[...]))
    qW = _mm(q, wq_ref[...])
    agts0 = _mm(x, agtw_ref[...])

    codes = codes_ref[0, 0, :]
    li = codes // N_ACT
    aj = codes % N_ACT
    e_iota_m = jax.lax.broadcasted_iota(jnp.int32, (E_BLK, M_BLK), 1)
    oh_li = (e_iota_m == li[:, None]).astype(jnp.bfloat16)
    e_iota_a = jax.lax.broadcasted_iota(jnp.int32, (E_BLK, N_ACT), 1)
    oh_aj = (e_iota_a == aj[:, None]).astype(jnp.bfloat16)

    s_q = _oh_mm(oh_li, qW)
    cw_e = _oh_mm(oh_aj, cw_ref[...])

    dx = dx_ref[0, 0, :][:, None]
    dy = dy_ref[0, 0, :][:, None]
    h = jax.nn.relu(dx * w1x_ref[...] + dy * w1y_ref[...] + b1_ref[...])
    d = jax.nn.relu(_gn(_mm(h, w2_ref[...]), g2_ref[...], be2_ref[...]))
    pre = _mm(d, wd_ref[...]) + s_q + cw_e
    cc = jax.nn.relu(_gn(pre, cg1_ref[...], cb1_ref[...]))
    ce = _mm(cc, cw2_ref[...])

    acc = agts0 + _oh_tmm(oh_li, ce)

    x2 = jax.nn.relu(acc)
    y = _gn(_mm(x2, linw_ref[...]), ling_ref[...], linb_ref[...])
    out_ref[...] = jax.nn.relu(y + res)


def _cw_body(actors_ref, wc0_ref, wc1_ref, cw0_ref, cw1_ref):
    a = actors_ref[...]
    cw0_ref[...] = _mm(a, wc0_ref[...])
    cw1_ref[...] = _mm(a, wc1_ref[...])


def _row(shape):
    return pl.BlockSpec(shape, lambda i: (0,) * len(shape))


def _layer_call(fuse_meta, agts, codes, dxs, dys, cw, pars):
    n_map = agts.shape[0]
    nb = n_map // M_BLK
    in_specs = [
        pl.BlockSpec((M_BLK, 128), lambda i: (i, 0)),
        pl.BlockSpec((1, 1, E_BLK), lambda i: (i, 0, 0)),
        pl.BlockSpec((1, 1, E_BLK), lambda i: (i, 0, 0)),
        pl.BlockSpec((1, 1, E_BLK), lambda i: (i, 0, 0)),
        _row((N_ACT, 128)),
    ] + [_row(p.shape) for p in pars]
    return pl.pallas_call(
        functools.partial(_layer_body, fuse_meta),
        grid=(nb,),
        in_specs=in_specs,
        out_specs=pl.BlockSpec((M_BLK, 128), lambda i: (i, 0)),
        out_shape=jax.ShapeDtypeStruct((n_map, 128), jnp.float32),
    )(agts, codes, dxs, dys, cw, *pars)


def _att_pars(p):
    r = lambda v: v.reshape(1, -1)
    ctx_w1 = p['ctx_w1']
    return (
        p['query_w'], r(p['query_g']), r(p['query_b']), ctx_w1[:, 128:256],
        r(p['dist_w1'][:, 0]), r(p['dist_w1'][:, 1]), r(p['dist_b1']),
        p['dist_w2'], r(p['dist_g2']), r(p['dist_be2']),
        ctx_w1[:, 0:128], r(p['ctx_g1']), r(p['ctx_b1']), p['ctx_w2'],
        p['agt_w'], p['lin_w'], r(p['lin_g']), r(p['lin_b']),
    )


def kernel(feat, map_ctrs, actors, actor_ctrs, params, map_idcs, actor_idcs):
    n_map = feat.shape[0]
    nb = n_map // M_BLK

    codes, dxs, dys = _discover(map_ctrs, actor_ctrs)

    cw0, cw1 = pl.pallas_call(
        _cw_body,
        out_shape=[jax.ShapeDtypeStruct((N_ACT, 128), jnp.float32)] * 2,
    )(actors, params['att0']['ctx_w1'][:, 256:],
      params['att1']['ctx_w1'][:, 256:])

    meta = (params['meta_w'], params['meta_g'].reshape(1, -1),
            params['meta_b'].reshape(1, -1))
    p0 = meta + _att_pars(params['att0'])
    p1 = meta + _att_pars(params['att1'])

    x = _layer_call(True, feat, codes, dxs, dys, cw0, p0)
    x = _layer_call(False, x, codes, dxs, dys, cw1, p1)
    return x
